# Initial kernel scaffold; baseline (speedup 1.0000x reference)
import functools

import jax
import jax.numpy as jnp
from jax import lax
from jax.experimental import pallas as pl
from jax.experimental.pallas import tpu as pltpu
from jax.experimental.pallas import tpu_sc as plsc

B = 16384
D = 32
DC = 2 * D
NC = 2
NS = 16
NW = NC * NS
BPW = B // NW
CHUNK = 128
NCHUNK = BPW // CHUNK

BLK = 2048
NB = B // BLK


def _sc_gather_body(uid_hbm, mid_hbm, ut_hbm, mt_hbm, uo_hbm, mo_hbm,
                    uid_v, mid_v, urows, mrows, sem):
  wid = lax.axis_index("s") * NC + lax.axis_index("c")
  base = wid * BPW
  pltpu.sync_copy(uid_hbm.at[pl.ds(wid * NCHUNK, NCHUNK)], uid_v)
  pltpu.sync_copy(mid_hbm.at[pl.ds(wid * NCHUNK, NCHUNK)], mid_v)
  copies = []
  for j in range(NCHUNK):
    sl = pl.ds(j * CHUNK, CHUNK)
    copies.append(pltpu.async_copy(ut_hbm.at[uid_v.at[j]], urows.at[sl], sem))
    copies.append(pltpu.async_copy(mt_hbm.at[mid_v.at[j]], mrows.at[sl], sem))
  for c in copies:
    c.wait()
  pltpu.sync_copy(urows, uo_hbm.at[pl.ds(base, BPW)])
  pltpu.sync_copy(mrows, mo_hbm.at[pl.ds(base, BPW)])


def _make_sc_gather():
  mesh = plsc.VectorSubcoreMesh(core_axis_name="c", subcore_axis_name="s",
                                num_cores=NC, num_subcores=NS)
  return pl.kernel(
      _sc_gather_body,
      out_type=[jax.ShapeDtypeStruct((B, DC), jnp.float32),
                jax.ShapeDtypeStruct((B, DC), jnp.float32)],
      mesh=mesh,
      scratch_types=[
          pltpu.VMEM((NCHUNK, CHUNK), jnp.int32),
          pltpu.VMEM((NCHUNK, CHUNK), jnp.int32),
          pltpu.VMEM((BPW, DC), jnp.float32),
          pltpu.VMEM((BPW, DC), jnp.float32),
          pltpu.SemaphoreType.DMA,
      ],
  )


def _tc_body(u_ref, m_ref, w0u_ref, w0m_ref, b0_ref, w1_ref, b1_ref,
             w2_ref, b2_ref, wn_ref, bn_ref, o_ref):
  u = u_ref[...]
  m = m_ref[...]
  gmf = u[:, :D] * m[:, :D]
  h = jnp.dot(u[:, D:], w0u_ref[...], preferred_element_type=jnp.float32)
  h += jnp.dot(m[:, D:], w0m_ref[...], preferred_element_type=jnp.float32)
  h = jnp.maximum(h + b0_ref[...], 0.0)
  h = jnp.maximum(
      jnp.dot(h, w1_ref[...], preferred_element_type=jnp.float32) + b1_ref[...],
      0.0)
  h = jnp.maximum(
      jnp.dot(h, w2_ref[...], preferred_element_type=jnp.float32) + b2_ref[...],
      0.0)
  wn = wn_ref[...]
  logit = jnp.sum(gmf * wn[:, :D], axis=1) + jnp.sum(h * wn[:, D:], axis=1)
  o_ref[...] = logit + bn_ref[0]


def _full(shape):
  return pl.BlockSpec(shape, lambda i: tuple(0 for _ in shape))


def _make_tc_dense():
  return pl.pallas_call(
      _tc_body,
      grid=(NB,),
      in_specs=[
          pl.BlockSpec((BLK, DC), lambda i: (i, 0)),
          pl.BlockSpec((BLK, DC), lambda i: (i, 0)),
          _full((D, 128)),
          _full((D, 128)),
          _full((1, 128)),
          _full((128, 64)),
          _full((1, 64)),
          _full((64, D)),
          _full((1, D)),
          _full((1, DC)),
          pl.BlockSpec(memory_space=pltpu.SMEM),
      ],
      out_specs=pl.BlockSpec((BLK,), lambda i: (i,)),
      out_shape=jax.ShapeDtypeStruct((B,), jnp.float32),
      compiler_params=pltpu.CompilerParams(
          dimension_semantics=("arbitrary",)),
  )


@jax.jit
def kernel(user_id, movie_title, user_gmf, movie_gmf, user_mlp, movie_mlp,
           W0, b0, W1, b1, W2, b2, Wn, bn):
  ut = jnp.concatenate([user_gmf, user_mlp], axis=1)
  mt = jnp.concatenate([movie_gmf, movie_mlp], axis=1)
  uid = user_id.astype(jnp.int32).reshape(NW * NCHUNK, CHUNK)
  mid = movie_title.astype(jnp.int32).reshape(NW * NCHUNK, CHUNK)
  urows, mrows = _make_sc_gather()(uid, mid, ut, mt)
  out = _make_tc_dense()(
      urows, mrows, W0[:D], W0[D:], b0.reshape(1, 128), W1,
      b1.reshape(1, 64), W2, b2.reshape(1, D), Wn.reshape(1, DC), bn)
  return out


# speedup vs baseline: 3.6881x; 3.6881x over previous
import functools

import jax
import jax.numpy as jnp
from jax import lax
from jax.experimental import pallas as pl
from jax.experimental.pallas import tpu as pltpu
from jax.experimental.pallas import tpu_sc as plsc

B = 16384
D = 32
DC = 2 * D
NC = 2
NS = 16
NW = NC * NS
BPW = B // NW
CHUNK = 128
NCHUNK = BPW // CHUNK

BLK = 2048
NB = B // BLK


def _sc_gather_body(uid_hbm, mid_hbm, ut_hbm, mt_hbm, uo_hbm, mo_hbm,
                    uid_v, mid_v, urows, mrows, sem):
  wid = lax.axis_index("s") * NC + lax.axis_index("c")
  base = wid * BPW
  pltpu.sync_copy(uid_hbm.at[pl.ds(wid * NCHUNK, NCHUNK)], uid_v)
  pltpu.sync_copy(mid_hbm.at[pl.ds(wid * NCHUNK, NCHUNK)], mid_v)
  copies = []
  for j in range(NCHUNK):
    sl = pl.ds(j * CHUNK, CHUNK)
    copies.append(pltpu.async_copy(ut_hbm.at[uid_v.at[j]], urows.at[sl], sem))
    copies.append(pltpu.async_copy(mt_hbm.at[mid_v.at[j]], mrows.at[sl], sem))
  for c in copies:
    c.wait()
  pltpu.sync_copy(urows, uo_hbm.at[pl.ds(base, BPW)])
  pltpu.sync_copy(mrows, mo_hbm.at[pl.ds(base, BPW)])


def _make_sc_gather():
  mesh = plsc.VectorSubcoreMesh(core_axis_name="c", subcore_axis_name="s",
                                num_cores=NC, num_subcores=NS)
  return pl.kernel(
      _sc_gather_body,
      out_type=[jax.ShapeDtypeStruct((B, DC), jnp.float32),
                jax.ShapeDtypeStruct((B, DC), jnp.float32)],
      mesh=mesh,
      scratch_types=[
          pltpu.VMEM((NCHUNK, CHUNK), jnp.int32),
          pltpu.VMEM((NCHUNK, CHUNK), jnp.int32),
          pltpu.VMEM((BPW, DC), jnp.float32),
          pltpu.VMEM((BPW, DC), jnp.float32),
          pltpu.SemaphoreType.DMA,
      ],
      compiler_params=pltpu.CompilerParams(use_tc_tiling_on_sc=False),
  )


def _tc_body(u_ref, m_ref, w0u_ref, w0m_ref, b0_ref, w1_ref, b1_ref,
             w2_ref, b2_ref, wn_ref, bn_ref, o_ref):
  u = u_ref[...]
  m = m_ref[...]
  gmf = u[:, :D] * m[:, :D]
  h = jnp.dot(u[:, D:], w0u_ref[...], preferred_element_type=jnp.float32)
  h += jnp.dot(m[:, D:], w0m_ref[...], preferred_element_type=jnp.float32)
  h = jnp.maximum(h + b0_ref[...], 0.0)
  h = jnp.maximum(
      jnp.dot(h, w1_ref[...], preferred_element_type=jnp.float32) + b1_ref[...],
      0.0)
  h = jnp.maximum(
      jnp.dot(h, w2_ref[...], preferred_element_type=jnp.float32) + b2_ref[...],
      0.0)
  wn = wn_ref[...]
  logit = jnp.sum(gmf * wn[:, :D], axis=1) + jnp.sum(h * wn[:, D:], axis=1)
  o_ref[...] = logit + bn_ref[0]


def _full(shape):
  return pl.BlockSpec(shape, lambda i: tuple(0 for _ in shape))


def _make_tc_dense():
  return pl.pallas_call(
      _tc_body,
      grid=(NB,),
      in_specs=[
          pl.BlockSpec((BLK, DC), lambda i: (i, 0)),
          pl.BlockSpec((BLK, DC), lambda i: (i, 0)),
          _full((D, 128)),
          _full((D, 128)),
          _full((1, 128)),
          _full((128, 64)),
          _full((1, 64)),
          _full((64, D)),
          _full((1, D)),
          _full((1, DC)),
          pl.BlockSpec(memory_space=pltpu.SMEM),
      ],
      out_specs=pl.BlockSpec((BLK,), lambda i: (i,)),
      out_shape=jax.ShapeDtypeStruct((B,), jnp.float32),
      compiler_params=pltpu.CompilerParams(
          dimension_semantics=("arbitrary",)),
  )


@jax.jit
def kernel(user_id, movie_title, user_gmf, movie_gmf, user_mlp, movie_mlp,
           W0, b0, W1, b1, W2, b2, Wn, bn):
  ut = jnp.concatenate([user_gmf, user_mlp], axis=1)
  mt = jnp.concatenate([movie_gmf, movie_mlp], axis=1)
  uid = user_id.astype(jnp.int32).reshape(NW * NCHUNK, CHUNK)
  mid = movie_title.astype(jnp.int32).reshape(NW * NCHUNK, CHUNK)
  urows, mrows = _make_sc_gather()(uid, mid, ut, mt)
  out = _make_tc_dense()(
      urows, mrows, W0[:D], W0[D:], b0.reshape(1, 128), W1,
      b1.reshape(1, 64), W2, b2.reshape(1, D), Wn.reshape(1, DC), bn)
  return out


# speedup vs baseline: 4.4682x; 1.2115x over previous
import functools

import jax
import jax.numpy as jnp
from jax import lax
from jax.experimental import pallas as pl
from jax.experimental.pallas import tpu as pltpu
from jax.experimental.pallas import tpu_sc as plsc

B = 16384
D = 32
DC = 2 * D
NC = 2
NS = 16
NW = NC * NS
BPW = B // NW
CHUNK = 128
NCHUNK = BPW // CHUNK

BLK = 2048
NB = B // BLK


def _sc_gather_body(uid_hbm, mid_hbm, ut_hbm, mt_hbm, o_hbm,
                    uid_v, mid_v, urows, mrows, sem):
  wid = lax.axis_index("s") * NC + lax.axis_index("c")
  base = wid * BPW
  pltpu.sync_copy(uid_hbm.at[pl.ds(wid * NCHUNK, NCHUNK)], uid_v)
  pltpu.sync_copy(mid_hbm.at[pl.ds(wid * NCHUNK, NCHUNK)], mid_v)
  copies = []
  for j in range(NCHUNK):
    sl = pl.ds(j * CHUNK, CHUNK)
    copies.append(pltpu.async_copy(ut_hbm.at[uid_v.at[j]], urows.at[sl], sem))
    copies.append(pltpu.async_copy(mt_hbm.at[mid_v.at[j]], mrows.at[sl], sem))
  for c in copies:
    c.wait()
  pltpu.sync_copy(urows, o_hbm.at[pl.ds(base, BPW), pl.ds(0, DC)])
  pltpu.sync_copy(mrows, o_hbm.at[pl.ds(base, BPW), pl.ds(DC, DC)])


def _make_sc_gather():
  mesh = plsc.VectorSubcoreMesh(core_axis_name="c", subcore_axis_name="s",
                                num_cores=NC, num_subcores=NS)
  return pl.kernel(
      _sc_gather_body,
      out_type=jax.ShapeDtypeStruct((B, 2 * DC), jnp.float32),
      mesh=mesh,
      scratch_types=[
          pltpu.VMEM((NCHUNK, CHUNK), jnp.int32),
          pltpu.VMEM((NCHUNK, CHUNK), jnp.int32),
          pltpu.VMEM((BPW, DC), jnp.float32),
          pltpu.VMEM((BPW, DC), jnp.float32),
          pltpu.SemaphoreType.DMA,
      ],
      compiler_params=pltpu.CompilerParams(use_tc_tiling_on_sc=False),
  )


def _tc_body(x_ref, w0u_ref, w0m_ref, b0_ref, w1_ref, b1_ref,
             w2_ref, b2_ref, wn_ref, bn_ref, o_ref):
  x = x_ref[...]
  gmf = x[:, :D] * x[:, DC:DC + D]
  h = jnp.dot(x[:, D:DC], w0u_ref[...], preferred_element_type=jnp.float32)
  h += jnp.dot(x[:, DC + D:], w0m_ref[...], preferred_element_type=jnp.float32)
  h = jnp.maximum(h + b0_ref[...], 0.0)
  h = jnp.maximum(
      jnp.dot(h, w1_ref[...], preferred_element_type=jnp.float32) + b1_ref[...],
      0.0)
  h = jnp.maximum(
      jnp.dot(h, w2_ref[...], preferred_element_type=jnp.float32) + b2_ref[...],
      0.0)
  wn = wn_ref[...]
  logit = jnp.sum(gmf * wn[:, :D], axis=1) + jnp.sum(h * wn[:, D:], axis=1)
  o_ref[...] = logit + bn_ref[0]


def _full(shape):
  return pl.BlockSpec(shape, lambda i: tuple(0 for _ in shape))


def _make_tc_dense():
  return pl.pallas_call(
      _tc_body,
      grid=(NB,),
      in_specs=[
          pl.BlockSpec((BLK, 2 * DC), lambda i: (i, 0)),
          _full((D, 128)),
          _full((D, 128)),
          _full((1, 128)),
          _full((128, 64)),
          _full((1, 64)),
          _full((64, D)),
          _full((1, D)),
          _full((1, DC)),
          pl.BlockSpec(memory_space=pltpu.SMEM),
      ],
      out_specs=pl.BlockSpec((BLK,), lambda i: (i,)),
      out_shape=jax.ShapeDtypeStruct((B,), jnp.float32),
      compiler_params=pltpu.CompilerParams(
          dimension_semantics=("arbitrary",)),
  )


@jax.jit
def kernel(user_id, movie_title, user_gmf, movie_gmf, user_mlp, movie_mlp,
           W0, b0, W1, b1, W2, b2, Wn, bn):
  ut = jnp.concatenate([user_gmf, user_mlp], axis=1)
  mt = jnp.concatenate([movie_gmf, movie_mlp], axis=1)
  uid = user_id.astype(jnp.int32).reshape(NW * NCHUNK, CHUNK)
  mid = movie_title.astype(jnp.int32).reshape(NW * NCHUNK, CHUNK)
  rows = _make_sc_gather()(uid, mid, ut, mt)
  out = _make_tc_dense()(
      rows, W0[:D], W0[D:], b0.reshape(1, 128), W1,
      b1.reshape(1, 64), W2, b2.reshape(1, D), Wn.reshape(1, DC), bn)
  return out
